# baseline (speedup 1.0000x reference)
import functools

import jax
import jax.numpy as jnp
from jax import lax
from jax.experimental import pallas as pl
from jax.experimental.pallas import tpu as pltpu
from jax.experimental.pallas import tpu_sc as plsc

N_EMB = 8192
DIM = 64
BETA = 0.25
BZ = 512
CE = 1024


def _argmin_body(z_ref, emb_ref, idx_ref, loss_ref, enorm_ref):
    @pl.when(pl.program_id(0) == 0)
    def _fill():
        embv = emb_ref[...]
        enorm_ref[...] = jnp.sum(embv * embv, axis=1, keepdims=True)

    zb = z_ref[...]
    zb2 = zb + zb
    zt = zb.T
    s = zt * zt
    h = DIM // 2
    while h >= 1:
        s = s[:h] + s[h:]
        h //= 2
    znr = s
    st = [[jnp.full((8, BZ), jnp.inf, jnp.float32),
           jnp.zeros((8, BZ), jnp.int32)] for _ in range(2)]
    ng = CE // 8
    for c in range(N_EMB // CE):
        eb = emb_ref[pl.ds(c * CE, CE), :]
        mm2 = lax.dot_general(eb, zb2, (((1,), (1,)), ((), ())),
                              preferred_element_type=jnp.float32)
        t1 = enorm_ref[pl.ds(c * CE, CE), :] + znr
        d = t1 - mm2
        for k in range(0, ng, 2):
            d1 = d[8 * k:8 * k + 8, :]
            d2 = d[8 * k + 8:8 * k + 16, :]
            which = d2 < d1
            mp = jnp.minimum(d1, d2)
            ap = jnp.where(which, c * ng + k + 1, c * ng + k)
            ms, As = st[(k // 2) % 2]
            upd = mp < ms
            st[(k // 2) % 2] = [jnp.where(upd, mp, ms), jnp.where(upd, ap, As)]
    (m, a), (m1, a1) = st
    mg = (m1 < m) | ((m1 == m) & (a1 < a))
    m = jnp.where(mg, m1, m)
    a = jnp.where(mg, a1, a)
    p = a * 8 + lax.broadcasted_iota(jnp.int32, (8, BZ), 0)
    h = 4
    while h >= 1:
        m_lo, m_hi = m[:h], m[h:]
        p_lo, p_hi = p[:h], p[h:]
        upd = (m_hi < m_lo) | ((m_hi == m_lo) & (p_hi < p_lo))
        m = jnp.where(upd, m_hi, m_lo)
        p = jnp.where(upd, p_hi, p_lo)
        h //= 2
    idx_ref[...] = p.reshape(BZ)

    @pl.when(pl.program_id(0) == 0)
    def _init():
        loss_ref[...] = jnp.zeros((1, BZ), jnp.float32)

    loss_ref[...] += m


def _argmin_call(z, emb):
    nz = z.shape[0] // BZ
    return pl.pallas_call(
        _argmin_body,
        grid=(nz,),
        in_specs=[pl.BlockSpec((BZ, DIM), lambda i: (i, 0)),
                  pl.BlockSpec((N_EMB, DIM), lambda i: (0, 0))],
        out_specs=[pl.BlockSpec((BZ,), lambda i: (i,)),
                   pl.BlockSpec((1, BZ), lambda i: (0, 0))],
        out_shape=[jax.ShapeDtypeStruct((z.shape[0],), jnp.int32),
                   jax.ShapeDtypeStruct((1, BZ), jnp.float32)],
        scratch_shapes=[pltpu.VMEM((N_EMB, 1), jnp.float32)],
    )(z, emb)


def _gather_call(emb, idx):
    B = idx.shape[0]
    info = plsc.get_sparse_core_info()
    nw = info.num_cores * info.num_subcores
    b_per_w = B // nw
    mesh = plsc.VectorSubcoreMesh(core_axis_name="c", subcore_axis_name="s")

    @functools.partial(
        pl.kernel, mesh=mesh,
        compiler_params=pltpu.CompilerParams(use_tc_tiling_on_sc=False),
        out_type=jax.ShapeDtypeStruct((B, DIM), jnp.float32),
        scratch_types=[
            pltpu.VMEM((b_per_w,), jnp.int32),
            pltpu.VMEM((b_per_w, DIM), jnp.float32),
            pltpu.SemaphoreType.DMA,
        ],
    )
    def gather_k(table_hbm, idx_hbm, out_hbm, idx_v, rows_v, sem):
        wid = lax.axis_index("s") * info.num_cores + lax.axis_index("c")
        base = wid * b_per_w
        pltpu.sync_copy(idx_hbm.at[pl.ds(base, b_per_w)], idx_v)
        pltpu.async_copy(table_hbm.at[idx_v], rows_v, sem).wait()
        pltpu.sync_copy(rows_v, out_hbm.at[pl.ds(base, b_per_w)])

    return gather_k(emb, idx)


def kernel(z, emb):
    idx, loss_acc = _argmin_call(z, emb)
    z_q = _gather_call(emb, idx)
    loss = BETA * (jnp.sum(loss_acc) / (z.shape[0] * DIM))
    z_q_st = z + (z_q - z)
    return (z_q_st, loss, loss, idx)


# speedup vs baseline: 1.0026x; 1.0026x over previous
import functools

import jax
import jax.numpy as jnp
from jax import lax
from jax.experimental import pallas as pl
from jax.experimental.pallas import tpu as pltpu
from jax.experimental.pallas import tpu_sc as plsc

N_EMB = 8192
DIM = 64
BETA = 0.25
BZ = 512
CE = 1024


def _argmin_body(z_ref, emb_ref, idx_ref, loss_ref, enorm_ref):
    @pl.when(pl.program_id(0) == 0)
    def _fill():
        embv = emb_ref[...]
        enorm_ref[...] = jnp.sum(embv * embv, axis=1, keepdims=True)

    zb = z_ref[...]
    zb2 = zb + zb
    zt = zb.T
    s = zt * zt
    h = DIM // 2
    while h >= 1:
        s = s[:h] + s[h:]
        h //= 2
    znr = s
    m = jnp.full((8, BZ), jnp.inf, jnp.float32)
    a = jnp.zeros((8, BZ), jnp.int32)
    ng = CE // 8
    for c in range(N_EMB // CE):
        eb = emb_ref[pl.ds(c * CE, CE), :]
        mm2 = lax.dot_general(eb, zb2, (((1,), (1,)), ((), ())),
                              preferred_element_type=jnp.float32)
        t1 = enorm_ref[pl.ds(c * CE, CE), :] + znr
        d = t1 - mm2
        for k in range(0, ng, 2):
            d1 = d[8 * k:8 * k + 8, :]
            d2 = d[8 * k + 8:8 * k + 16, :]
            which = d2 < d1
            mp = jnp.minimum(d1, d2)
            ap = jnp.where(which, c * ng + k + 1, c * ng + k)
            upd = mp < m
            m = jnp.where(upd, mp, m)
            a = jnp.where(upd, ap, a)
    p = a * 8 + lax.broadcasted_iota(jnp.int32, (8, BZ), 0)
    h = 4
    while h >= 1:
        m_lo, m_hi = m[:h], m[h:]
        p_lo, p_hi = p[:h], p[h:]
        upd = (m_hi < m_lo) | ((m_hi == m_lo) & (p_hi < p_lo))
        m = jnp.where(upd, m_hi, m_lo)
        p = jnp.where(upd, p_hi, p_lo)
        h //= 2
    idx_ref[...] = p.reshape(BZ)

    @pl.when(pl.program_id(0) == 0)
    def _init():
        loss_ref[...] = jnp.zeros((1, BZ), jnp.float32)

    loss_ref[...] += m


def _argmin_call(z, emb, block_off=0, nblocks=None):
    if nblocks is None:
        nblocks = z.shape[0] // BZ
    return pl.pallas_call(
        _argmin_body,
        grid=(nblocks,),
        in_specs=[pl.BlockSpec((BZ, DIM), lambda i: (i + block_off, 0)),
                  pl.BlockSpec((N_EMB, DIM), lambda i: (0, 0))],
        out_specs=[pl.BlockSpec((BZ,), lambda i: (i,)),
                   pl.BlockSpec((1, BZ), lambda i: (0, 0))],
        out_shape=[jax.ShapeDtypeStruct((nblocks * BZ,), jnp.int32),
                   jax.ShapeDtypeStruct((1, BZ), jnp.float32)],
        scratch_shapes=[pltpu.VMEM((N_EMB, 1), jnp.float32)],
    )(z, emb)


def _gather_call(emb, idx):
    B = idx.shape[0]
    info = plsc.get_sparse_core_info()
    nw = info.num_cores * info.num_subcores
    b_per_w = B // nw
    mesh = plsc.VectorSubcoreMesh(core_axis_name="c", subcore_axis_name="s")

    @functools.partial(
        pl.kernel, mesh=mesh,
        compiler_params=pltpu.CompilerParams(use_tc_tiling_on_sc=False),
        out_type=jax.ShapeDtypeStruct((B, DIM), jnp.float32),
        scratch_types=[
            pltpu.VMEM((b_per_w,), jnp.int32),
            pltpu.VMEM((b_per_w, DIM), jnp.float32),
            pltpu.SemaphoreType.DMA,
        ],
    )
    def gather_k(table_hbm, idx_hbm, out_hbm, idx_v, rows_v, sem):
        wid = lax.axis_index("s") * info.num_cores + lax.axis_index("c")
        base = wid * b_per_w
        pltpu.sync_copy(idx_hbm.at[pl.ds(base, b_per_w)], idx_v)
        pltpu.async_copy(table_hbm.at[idx_v], rows_v, sem).wait()
        pltpu.sync_copy(rows_v, out_hbm.at[pl.ds(base, b_per_w)])

    return gather_k(emb, idx)


def kernel(z, emb):
    n = z.shape[0]
    nb = n // BZ
    idxs, zqs, laccs = [], [], []
    for part in range(2):
        idx_p, lacc_p = _argmin_call(z, emb, block_off=part * (nb // 2),
                                     nblocks=nb // 2)
        zqs.append(_gather_call(emb, idx_p))
        idxs.append(idx_p)
        laccs.append(lacc_p)
    idx = jnp.concatenate(idxs)
    z_q = jnp.concatenate(zqs)
    loss = BETA * ((jnp.sum(laccs[0]) + jnp.sum(laccs[1])) / (n * DIM))
    z_q_st = z + (z_q - z)
    return (z_q_st, loss, loss, idx)


# speedup vs baseline: 1.0089x; 1.0062x over previous
import functools

import jax
import jax.numpy as jnp
from jax import lax
from jax.experimental import pallas as pl
from jax.experimental.pallas import tpu as pltpu
from jax.experimental.pallas import tpu_sc as plsc

N_EMB = 8192
DIM = 64
BETA = 0.25
BZ = 1024
CE = 1024


def _argmin_body(z_ref, emb_ref, idx_ref, loss_ref, enorm_ref):
    @pl.when(pl.program_id(0) == 0)
    def _fill():
        embv = emb_ref[...]
        enorm_ref[...] = jnp.sum(embv * embv, axis=1, keepdims=True)

    zb = z_ref[...]
    zb2 = zb + zb
    zt = zb.T
    s = zt * zt
    h = DIM // 2
    while h >= 1:
        s = s[:h] + s[h:]
        h //= 2
    znr = s
    m = jnp.full((8, BZ), jnp.inf, jnp.float32)
    a = jnp.zeros((8, BZ), jnp.int32)
    ng = CE // 8
    for c in range(N_EMB // CE):
        eb = emb_ref[pl.ds(c * CE, CE), :]
        mm2 = lax.dot_general(eb, zb2, (((1,), (1,)), ((), ())),
                              preferred_element_type=jnp.float32)
        t1 = enorm_ref[pl.ds(c * CE, CE), :] + znr
        d = t1 - mm2
        for k in range(0, ng, 2):
            d1 = d[8 * k:8 * k + 8, :]
            d2 = d[8 * k + 8:8 * k + 16, :]
            which = d2 < d1
            mp = jnp.minimum(d1, d2)
            ap = jnp.where(which, c * ng + k + 1, c * ng + k)
            upd = mp < m
            m = jnp.where(upd, mp, m)
            a = jnp.where(upd, ap, a)
    p = a * 8 + lax.broadcasted_iota(jnp.int32, (8, BZ), 0)
    h = 4
    while h >= 1:
        m_lo, m_hi = m[:h], m[h:]
        p_lo, p_hi = p[:h], p[h:]
        upd = (m_hi < m_lo) | ((m_hi == m_lo) & (p_hi < p_lo))
        m = jnp.where(upd, m_hi, m_lo)
        p = jnp.where(upd, p_hi, p_lo)
        h //= 2
    idx_ref[...] = p.reshape(BZ)

    @pl.when(pl.program_id(0) == 0)
    def _init():
        loss_ref[...] = jnp.zeros((1, BZ), jnp.float32)

    loss_ref[...] += m


def _argmin_call(z, emb, block_off=0, nblocks=None):
    if nblocks is None:
        nblocks = z.shape[0] // BZ
    return pl.pallas_call(
        _argmin_body,
        grid=(nblocks,),
        in_specs=[pl.BlockSpec((BZ, DIM), lambda i: (i + block_off, 0)),
                  pl.BlockSpec((N_EMB, DIM), lambda i: (0, 0))],
        out_specs=[pl.BlockSpec((BZ,), lambda i: (i,)),
                   pl.BlockSpec((1, BZ), lambda i: (0, 0))],
        out_shape=[jax.ShapeDtypeStruct((nblocks * BZ,), jnp.int32),
                   jax.ShapeDtypeStruct((1, BZ), jnp.float32)],
        scratch_shapes=[pltpu.VMEM((N_EMB, 1), jnp.float32)],
    )(z, emb)


def _gather_call(emb, idx):
    B = idx.shape[0]
    info = plsc.get_sparse_core_info()
    nw = info.num_cores * info.num_subcores
    b_per_w = B // nw
    mesh = plsc.VectorSubcoreMesh(core_axis_name="c", subcore_axis_name="s")

    @functools.partial(
        pl.kernel, mesh=mesh,
        compiler_params=pltpu.CompilerParams(use_tc_tiling_on_sc=False),
        out_type=jax.ShapeDtypeStruct((B, DIM), jnp.float32),
        scratch_types=[
            pltpu.VMEM((b_per_w,), jnp.int32),
            pltpu.VMEM((b_per_w, DIM), jnp.float32),
            pltpu.SemaphoreType.DMA,
        ],
    )
    def gather_k(table_hbm, idx_hbm, out_hbm, idx_v, rows_v, sem):
        wid = lax.axis_index("s") * info.num_cores + lax.axis_index("c")
        base = wid * b_per_w
        pltpu.sync_copy(idx_hbm.at[pl.ds(base, b_per_w)], idx_v)
        pltpu.async_copy(table_hbm.at[idx_v], rows_v, sem).wait()
        pltpu.sync_copy(rows_v, out_hbm.at[pl.ds(base, b_per_w)])

    return gather_k(emb, idx)


def kernel(z, emb):
    idx, loss_acc = _argmin_call(z, emb)
    z_q = _gather_call(emb, idx)
    loss = BETA * (jnp.sum(loss_acc) / (z.shape[0] * DIM))
    z_q_st = z + (z_q - z)
    return (z_q_st, loss, loss, idx)


# speedup vs baseline: 1.0236x; 1.0146x over previous
import functools

import jax
import jax.numpy as jnp
from jax import lax
from jax.experimental import pallas as pl
from jax.experimental.pallas import tpu as pltpu
from jax.experimental.pallas import tpu_sc as plsc

N_EMB = 8192
DIM = 64
BETA = 0.25
BZ = 2048
CE = 1024


def _argmin_body(z_ref, emb_ref, idx_ref, loss_ref, enorm_ref):
    @pl.when(pl.program_id(0) == 0)
    def _fill():
        embv = emb_ref[...]
        enorm_ref[...] = jnp.sum(embv * embv, axis=1, keepdims=True)

    zb = z_ref[...]
    zb2 = zb + zb
    zt = zb.T
    s = zt * zt
    h = DIM // 2
    while h >= 1:
        s = s[:h] + s[h:]
        h //= 2
    znr = s
    m = jnp.full((8, BZ), jnp.inf, jnp.float32)
    a = jnp.zeros((8, BZ), jnp.int32)
    ng = CE // 8
    for c in range(N_EMB // CE):
        eb = emb_ref[pl.ds(c * CE, CE), :]
        mm2 = lax.dot_general(eb, zb2, (((1,), (1,)), ((), ())),
                              preferred_element_type=jnp.float32)
        t1 = enorm_ref[pl.ds(c * CE, CE), :] + znr
        d = t1 - mm2
        for k in range(0, ng, 2):
            d1 = d[8 * k:8 * k + 8, :]
            d2 = d[8 * k + 8:8 * k + 16, :]
            which = d2 < d1
            mp = jnp.minimum(d1, d2)
            ap = jnp.where(which, c * ng + k + 1, c * ng + k)
            upd = mp < m
            m = jnp.where(upd, mp, m)
            a = jnp.where(upd, ap, a)
    p = a * 8 + lax.broadcasted_iota(jnp.int32, (8, BZ), 0)
    h = 4
    while h >= 1:
        m_lo, m_hi = m[:h], m[h:]
        p_lo, p_hi = p[:h], p[h:]
        upd = (m_hi < m_lo) | ((m_hi == m_lo) & (p_hi < p_lo))
        m = jnp.where(upd, m_hi, m_lo)
        p = jnp.where(upd, p_hi, p_lo)
        h //= 2
    idx_ref[...] = p.reshape(BZ)

    @pl.when(pl.program_id(0) == 0)
    def _init():
        loss_ref[...] = jnp.zeros((1, BZ), jnp.float32)

    loss_ref[...] += m


def _argmin_call(z, emb, block_off=0, nblocks=None):
    if nblocks is None:
        nblocks = z.shape[0] // BZ
    return pl.pallas_call(
        _argmin_body,
        grid=(nblocks,),
        in_specs=[pl.BlockSpec((BZ, DIM), lambda i: (i + block_off, 0)),
                  pl.BlockSpec((N_EMB, DIM), lambda i: (0, 0))],
        out_specs=[pl.BlockSpec((BZ,), lambda i: (i,)),
                   pl.BlockSpec((1, BZ), lambda i: (0, 0))],
        out_shape=[jax.ShapeDtypeStruct((nblocks * BZ,), jnp.int32),
                   jax.ShapeDtypeStruct((1, BZ), jnp.float32)],
        scratch_shapes=[pltpu.VMEM((N_EMB, 1), jnp.float32)],
    )(z, emb)


def _gather_call(emb, idx):
    B = idx.shape[0]
    info = plsc.get_sparse_core_info()
    nw = info.num_cores * info.num_subcores
    b_per_w = B // nw
    mesh = plsc.VectorSubcoreMesh(core_axis_name="c", subcore_axis_name="s")

    @functools.partial(
        pl.kernel, mesh=mesh,
        compiler_params=pltpu.CompilerParams(use_tc_tiling_on_sc=False),
        out_type=jax.ShapeDtypeStruct((B, DIM), jnp.float32),
        scratch_types=[
            pltpu.VMEM((b_per_w,), jnp.int32),
            pltpu.VMEM((b_per_w, DIM), jnp.float32),
            pltpu.SemaphoreType.DMA,
        ],
    )
    def gather_k(table_hbm, idx_hbm, out_hbm, idx_v, rows_v, sem):
        wid = lax.axis_index("s") * info.num_cores + lax.axis_index("c")
        base = wid * b_per_w
        pltpu.sync_copy(idx_hbm.at[pl.ds(base, b_per_w)], idx_v)
        pltpu.async_copy(table_hbm.at[idx_v], rows_v, sem).wait()
        pltpu.sync_copy(rows_v, out_hbm.at[pl.ds(base, b_per_w)])

    return gather_k(emb, idx)


def kernel(z, emb):
    idx, loss_acc = _argmin_call(z, emb)
    z_q = _gather_call(emb, idx)
    loss = BETA * (jnp.sum(loss_acc) / (z.shape[0] * DIM))
    z_q_st = z + (z_q - z)
    return (z_q_st, loss, loss, idx)


# speedup vs baseline: 1.0375x; 1.0136x over previous
import functools

import jax
import jax.numpy as jnp
from jax import lax
from jax.experimental import pallas as pl
from jax.experimental.pallas import tpu as pltpu
from jax.experimental.pallas import tpu_sc as plsc

N_EMB = 8192
DIM = 64
BETA = 0.25
BZ = 4096
CE = 1024


def _argmin_body(z_ref, emb_ref, idx_ref, loss_ref, enorm_ref):
    @pl.when(pl.program_id(0) == 0)
    def _fill():
        embv = emb_ref[...]
        enorm_ref[...] = jnp.sum(embv * embv, axis=1, keepdims=True)

    zb = z_ref[...]
    zb2 = zb + zb
    zt = zb.T
    s = zt * zt
    h = DIM // 2
    while h >= 1:
        s = s[:h] + s[h:]
        h //= 2
    znr = s
    m = jnp.full((8, BZ), jnp.inf, jnp.float32)
    a = jnp.zeros((8, BZ), jnp.int32)
    ng = CE // 8
    for c in range(N_EMB // CE):
        eb = emb_ref[pl.ds(c * CE, CE), :]
        mm2 = lax.dot_general(eb, zb2, (((1,), (1,)), ((), ())),
                              preferred_element_type=jnp.float32)
        t1 = enorm_ref[pl.ds(c * CE, CE), :] + znr
        d = t1 - mm2
        for k in range(0, ng, 2):
            d1 = d[8 * k:8 * k + 8, :]
            d2 = d[8 * k + 8:8 * k + 16, :]
            which = d2 < d1
            mp = jnp.minimum(d1, d2)
            ap = jnp.where(which, c * ng + k + 1, c * ng + k)
            upd = mp < m
            m = jnp.where(upd, mp, m)
            a = jnp.where(upd, ap, a)
    p = a * 8 + lax.broadcasted_iota(jnp.int32, (8, BZ), 0)
    h = 4
    while h >= 1:
        m_lo, m_hi = m[:h], m[h:]
        p_lo, p_hi = p[:h], p[h:]
        upd = (m_hi < m_lo) | ((m_hi == m_lo) & (p_hi < p_lo))
        m = jnp.where(upd, m_hi, m_lo)
        p = jnp.where(upd, p_hi, p_lo)
        h //= 2
    idx_ref[...] = p.reshape(BZ)

    @pl.when(pl.program_id(0) == 0)
    def _init():
        loss_ref[...] = jnp.zeros((1, BZ), jnp.float32)

    loss_ref[...] += m


def _argmin_call(z, emb, block_off=0, nblocks=None):
    if nblocks is None:
        nblocks = z.shape[0] // BZ
    return pl.pallas_call(
        _argmin_body,
        grid=(nblocks,),
        in_specs=[pl.BlockSpec((BZ, DIM), lambda i: (i + block_off, 0)),
                  pl.BlockSpec((N_EMB, DIM), lambda i: (0, 0))],
        out_specs=[pl.BlockSpec((BZ,), lambda i: (i,)),
                   pl.BlockSpec((1, BZ), lambda i: (0, 0))],
        out_shape=[jax.ShapeDtypeStruct((nblocks * BZ,), jnp.int32),
                   jax.ShapeDtypeStruct((1, BZ), jnp.float32)],
        scratch_shapes=[pltpu.VMEM((N_EMB, 1), jnp.float32)],
    )(z, emb)


def _gather_call(emb, idx):
    B = idx.shape[0]
    info = plsc.get_sparse_core_info()
    nw = info.num_cores * info.num_subcores
    b_per_w = B // nw
    mesh = plsc.VectorSubcoreMesh(core_axis_name="c", subcore_axis_name="s")

    @functools.partial(
        pl.kernel, mesh=mesh,
        compiler_params=pltpu.CompilerParams(use_tc_tiling_on_sc=False),
        out_type=jax.ShapeDtypeStruct((B, DIM), jnp.float32),
        scratch_types=[
            pltpu.VMEM((b_per_w,), jnp.int32),
            pltpu.VMEM((b_per_w, DIM), jnp.float32),
            pltpu.SemaphoreType.DMA,
        ],
    )
    def gather_k(table_hbm, idx_hbm, out_hbm, idx_v, rows_v, sem):
        wid = lax.axis_index("s") * info.num_cores + lax.axis_index("c")
        base = wid * b_per_w
        pltpu.sync_copy(idx_hbm.at[pl.ds(base, b_per_w)], idx_v)
        pltpu.async_copy(table_hbm.at[idx_v], rows_v, sem).wait()
        pltpu.sync_copy(rows_v, out_hbm.at[pl.ds(base, b_per_w)])

    return gather_k(emb, idx)


def kernel(z, emb):
    idx, loss_acc = _argmin_call(z, emb)
    z_q = _gather_call(emb, idx)
    loss = BETA * (jnp.sum(loss_acc) / (z.shape[0] * DIM))
    z_q_st = z + (z_q - z)
    return (z_q_st, loss, loss, idx)
